# baseline (speedup 1.0000x reference)
import jax
import jax.numpy as jnp
from jax import lax
from jax.experimental import pallas as pl
from jax.experimental.pallas import tpu as pltpu
from jax.experimental.pallas import tpu_sc as plsc

_VOCAB = 50257
_DIM = 768
_BATCH = 64
_SEQ = 1024
_TOK = _BATCH * _SEQ

_NC = 2
_NS = 16
_NW = _NC * _NS
_TPW = _TOK // _NW
_LANES = 16
_CH = _LANES
_GRP = _TPW // _LANES
_ROWS = _GRP + 2
_BUF = _ROWS * _LANES
_DREGS = _DIM // _LANES
_ZWIN = 16
_NB = 4
_GDIST = 2


def _body(ids_hbm, mask_hbm, table_hbm, out_hbm,
          ids_c, uslot_c, mslot_c, uslot2, mslot2,
          rbuf, zbuf, gsems, osems, zsem):
    wid = lax.axis_index("c") * _NS + lax.axis_index("s")
    base = wid * _TPW
    zero16 = jnp.zeros((_LANES,), jnp.float32)

    pltpu.sync_copy(ids_hbm.at[wid], mslot2.at[pl.ds(0, _GRP)])
    pltpu.sync_copy(mask_hbm.at[wid], uslot2.at[pl.ds(0, _GRP)])

    def zrow(r, _):
        for j in range(_DREGS):
            zbuf[r, pl.ds(j * _LANES, _LANES)] = zero16
        return 0

    lax.fori_loop(0, _CH, zrow, 0, unroll=False)

    def part(g, carry):
        n1, n0 = carry
        id16 = mslot2[g, :]
        m16 = uslot2[g, :]
        slot0 = base + g * _LANES
        for t in range(_LANES):
            mt = m16[t]
            ids_c[pl.ds(n1, _LANES)] = jnp.full((_LANES,), id16[t], jnp.int32)
            uslot_c[pl.ds(n1, _LANES)] = jnp.full((_LANES,), slot0 + t,
                                                  jnp.int32)
            mslot_c[pl.ds(n0, _LANES)] = jnp.full((_LANES,), slot0 + t,
                                                  jnp.int32)
            n1 = n1 + mt
            n0 = n0 + (1 - mt)
        return n1, n0

    n1, n0 = lax.fori_loop(0, _GRP, part, (jnp.int32(0), jnp.int32(0)),
                           unroll=False)

    @pl.when(n1 > 0)
    def _():
        last_id = ids_c[pl.ds(n1 - 1, _LANES)][0]
        last_sl = uslot_c[pl.ds(n1 - 1, _LANES)][0]
        ids_c[pl.ds(n1, _LANES)] = jnp.full((_LANES,), last_id, jnp.int32)
        uslot_c[pl.ds(n1, _LANES)] = jnp.full((_LANES,), last_sl, jnp.int32)

    @pl.when(n0 > 0)
    def _():
        last_ms = mslot_c[pl.ds(n0 - 1, _LANES)][0]
        mslot_c[pl.ds(n0, _LANES)] = jnp.full((_LANES,), last_ms, jnp.int32)

    def relay(r, _):
        sl = pl.ds(r * _LANES, _LANES)
        uslot2[r, :] = uslot_c[sl]
        mslot2[r, :] = mslot_c[sl]
        return 0

    lax.fori_loop(0, _ROWS, relay, 0, unroll=False)

    c1 = (n1 + _CH - 1) // _CH
    c0 = (n0 + _CH - 1) // _CH

    def gdesc(k, b):
        return pltpu.make_async_copy(
            table_hbm.at[ids_c.at[pl.ds(k * _CH, _CH)]], rbuf.at[b],
            gsems[b])

    def sdesc(k, b):
        return pltpu.make_async_copy(rbuf.at[b], out_hbm.at[uslot2.at[k]],
                                     osems[b])

    def zdesc(k):
        return pltpu.make_async_copy(zbuf, out_hbm.at[mslot2.at[k]], zsem)

    for b in range(_GDIST):
        @pl.when(b < c1)
        def _(b=b):
            gdesc(b, b).start()

    def octo(q, _):
        for b in range(_NB):
            k = q * _NB + b
            b2 = (b + _GDIST) % _NB

            @pl.when(k < c1)
            def _(k=k, b=b, b2=b2):
                gdesc(k, b).wait()
                sdesc(k, b).start()

                @pl.when(k >= _GDIST)
                def _():
                    sdesc(k - _GDIST, b2).wait()

                @pl.when(k + _GDIST < c1)
                def _():
                    gdesc(k + _GDIST, b2).start()

            @pl.when(k < c0)
            def _(k=k):
                zdesc(k).start()

                @pl.when(k >= _ZWIN)
                def _():
                    zdesc(k - _ZWIN).wait()
        return 0

    nq = (jnp.maximum(c1, c0) + _NB - 1) // _NB
    lax.fori_loop(0, nq, octo, 0, unroll=False)

    for b in range(_NB):
        tail = jnp.bool_(False)
        for d in range(1, _GDIST + 1):
            tail = tail | ((c1 >= d) & ((c1 - d) % _NB == b))

        @pl.when(tail)
        def _(b=b):
            sdesc(0, b).wait()

    lax.fori_loop(0, jnp.minimum(c0, _ZWIN), lambda i, _: (zdesc(0).wait(), 0)[1],
                  0, unroll=False)


@jax.jit
def _lookup(ids, mask_i, table):
    mesh = plsc.VectorSubcoreMesh(core_axis_name="c", subcore_axis_name="s")
    run = pl.kernel(
        _body,
        out_type=jax.ShapeDtypeStruct((_TOK, _DIM), jnp.float32),
        mesh=mesh,
        scratch_types=[
            pltpu.VMEM((_BUF,), jnp.int32),
            pltpu.VMEM((_BUF,), jnp.int32),
            pltpu.VMEM((_BUF,), jnp.int32),
            pltpu.VMEM((_ROWS, _CH), jnp.int32),
            pltpu.VMEM((_ROWS, _CH), jnp.int32),
            pltpu.VMEM((_NB, _CH, _DIM), jnp.float32),
            pltpu.VMEM((_CH, _DIM), jnp.float32),
            [pltpu.SemaphoreType.DMA] * _NB,
            [pltpu.SemaphoreType.DMA] * _NB,
            pltpu.SemaphoreType.DMA,
        ],
    )
    return run(ids, mask_i, table)


def kernel(input_ids, attention_mask, table):
    ids = input_ids.reshape(_NW, _GRP, _LANES).astype(jnp.int32)
    mask_i = attention_mask.reshape(_NW, _GRP, _LANES).astype(jnp.int32)
    out = _lookup(ids, mask_i, table)
    return out.reshape(_BATCH, _SEQ, _DIM), attention_mask


# speedup vs baseline: 1.0028x; 1.0028x over previous
import jax
import jax.numpy as jnp
from jax import lax
from jax.experimental import pallas as pl
from jax.experimental.pallas import tpu as pltpu
from jax.experimental.pallas import tpu_sc as plsc

_VOCAB = 50257
_DIM = 768
_BATCH = 64
_SEQ = 1024
_TOK = _BATCH * _SEQ

_NC = 2
_NS = 16
_NW = _NC * _NS
_TPW = _TOK // _NW
_LANES = 16
_CH = _LANES
_GRP = _TPW // _LANES
_ROWS = _GRP + 2
_BUF = _ROWS * _LANES
_DREGS = _DIM // _LANES
_ZWIN = 16
_NB = 4
_GDIST = 2


def _body(ids_hbm, mask_hbm, table_hbm, out_hbm,
          ids_c, uslot_c, mslot_c, uslot2, mslot2,
          rbuf, zbuf, gsems, osems, zsem):
    wid = lax.axis_index("c") * _NS + lax.axis_index("s")
    base = wid * _TPW
    zero16 = jnp.zeros((_LANES,), jnp.float32)

    pltpu.sync_copy(ids_hbm.at[wid], mslot2.at[pl.ds(0, _GRP)])
    pltpu.sync_copy(mask_hbm.at[wid], uslot2.at[pl.ds(0, _GRP)])

    def zrow(r, _):
        for j in range(_DREGS):
            zbuf[r, pl.ds(j * _LANES, _LANES)] = zero16
        return 0

    lax.fori_loop(0, _CH, zrow, 0, unroll=False)

    def part(g, carry):
        n1, n0 = carry
        id16 = mslot2[g, :]
        m16 = uslot2[g, :]
        slot0 = base + g * _LANES
        for t in range(_LANES):
            mt = m16[t]
            ids_c[pl.ds(n1, _LANES)] = jnp.full((_LANES,), id16[t], jnp.int32)
            uslot_c[pl.ds(n1, _LANES)] = jnp.full((_LANES,), slot0 + t,
                                                  jnp.int32)
            mslot_c[pl.ds(n0, _LANES)] = jnp.full((_LANES,), slot0 + t,
                                                  jnp.int32)
            n1 = n1 + mt
            n0 = n0 + (1 - mt)
        return n1, n0

    n1, n0 = lax.fori_loop(0, _GRP, part, (jnp.int32(0), jnp.int32(0)),
                           unroll=False)

    @pl.when(n1 > 0)
    def _():
        last_id = ids_c[pl.ds(n1 - 1, _LANES)][0]
        last_sl = uslot_c[pl.ds(n1 - 1, _LANES)][0]
        ids_c[pl.ds(n1, _LANES)] = jnp.full((_LANES,), last_id, jnp.int32)
        uslot_c[pl.ds(n1, _LANES)] = jnp.full((_LANES,), last_sl, jnp.int32)

    @pl.when(n0 > 0)
    def _():
        last_ms = mslot_c[pl.ds(n0 - 1, _LANES)][0]
        mslot_c[pl.ds(n0, _LANES)] = jnp.full((_LANES,), last_ms, jnp.int32)

    def relay(r, _):
        sl = pl.ds(r * _LANES, _LANES)
        uslot2[r, :] = uslot_c[sl]
        mslot2[r, :] = mslot_c[sl]
        return 0

    lax.fori_loop(0, _ROWS, relay, 0, unroll=False)

    c1 = (n1 + _CH - 1) // _CH
    c0 = (n0 + _CH - 1) // _CH

    def gdesc(k, b):
        return pltpu.make_async_copy(
            table_hbm.at[ids_c.at[pl.ds(k * _CH, _CH)]], rbuf.at[b],
            gsems.at[b])

    def sdesc(k, b):
        return pltpu.make_async_copy(rbuf.at[b], out_hbm.at[uslot2.at[k]],
                                     osems.at[b])

    def zdesc(k):
        return pltpu.make_async_copy(zbuf, out_hbm.at[mslot2.at[k]], zsem)

    def prime(k, _):
        gdesc(k, k % _NB).start()
        return 0

    lax.fori_loop(0, jnp.minimum(c1, _GDIST), prime, 0, unroll=False)

    def step(k, _):
        b = k % _NB
        b2 = (k + _GDIST) % _NB

        @pl.when(k < c1)
        def _():
            gdesc(k, b).wait()
            sdesc(k, b).start()

            @pl.when(k >= _GDIST)
            def _():
                sdesc(k - _GDIST, b2).wait()

            @pl.when(k + _GDIST < c1)
            def _():
                gdesc(k + _GDIST, b2).start()

        @pl.when(k < c0)
        def _():
            zdesc(k).start()

            @pl.when(k >= _ZWIN)
            def _():
                zdesc(k - _ZWIN).wait()
        return 0

    lax.fori_loop(0, jnp.maximum(c1, c0), step, 0, unroll=False)

    def draink(i, _):
        k = jnp.maximum(c1 - _GDIST, 0) + i

        @pl.when(k < c1)
        def _():
            sdesc(0, k % _NB).wait()
        return 0

    lax.fori_loop(0, _GDIST, draink, 0, unroll=False)

    lax.fori_loop(0, jnp.minimum(c0, _ZWIN), lambda i, _: (zdesc(0).wait(), 0)[1],
                  0, unroll=False)


@jax.jit
def _lookup(ids, mask_i, table):
    mesh = plsc.VectorSubcoreMesh(core_axis_name="c", subcore_axis_name="s")
    run = pl.kernel(
        _body,
        out_type=jax.ShapeDtypeStruct((_TOK, _DIM), jnp.float32),
        mesh=mesh,
        scratch_types=[
            pltpu.VMEM((_BUF,), jnp.int32),
            pltpu.VMEM((_BUF,), jnp.int32),
            pltpu.VMEM((_BUF,), jnp.int32),
            pltpu.VMEM((_ROWS, _CH), jnp.int32),
            pltpu.VMEM((_ROWS, _CH), jnp.int32),
            pltpu.VMEM((_NB, _CH, _DIM), jnp.float32),
            pltpu.VMEM((_CH, _DIM), jnp.float32),
            pltpu.SemaphoreType.DMA((_NB,)),
            pltpu.SemaphoreType.DMA((_NB,)),
            pltpu.SemaphoreType.DMA,
        ],
    )
    return run(ids, mask_i, table)


def kernel(input_ids, attention_mask, table):
    ids = input_ids.reshape(_NW, _GRP, _LANES).astype(jnp.int32)
    mask_i = attention_mask.reshape(_NW, _GRP, _LANES).astype(jnp.int32)
    out = _lookup(ids, mask_i, table)
    return out.reshape(_BATCH, _SEQ, _DIM), attention_mask


# speedup vs baseline: 1.0347x; 1.0318x over previous
import jax
import jax.numpy as jnp
from jax import lax
from jax.experimental import pallas as pl
from jax.experimental.pallas import tpu as pltpu
from jax.experimental.pallas import tpu_sc as plsc

_VOCAB = 50257
_DIM = 768
_BATCH = 64
_SEQ = 1024
_TOK = _BATCH * _SEQ

_NC = 2
_NS = 16
_NW = _NC * _NS
_TPW = _TOK // _NW
_LANES = 16
_CH = _LANES
_GRP = _TPW // _LANES
_ROWS = _GRP + 2
_BUF = _ROWS * _LANES
_DREGS = _DIM // _LANES
_ZWIN = 16
_NB = 4
_GDIST = 2
_BLK = 16
_NBLK = _GRP // _BLK


def _body(ids_hbm, mask_hbm, table_hbm, out_hbm,
          ids_c, uslot_c, mslot_c, uslot2, mslot2,
          rbuf, zbuf, gsems, osems, zsem):
    wid = lax.axis_index("c") * _NS + lax.axis_index("s")
    base = wid * _TPW
    zero16 = jnp.zeros((_LANES,), jnp.float32)

    pltpu.sync_copy(ids_hbm.at[wid], mslot2.at[pl.ds(0, _GRP)])
    pltpu.sync_copy(mask_hbm.at[wid], uslot2.at[pl.ds(0, _GRP)])

    def zrow(r, _):
        for j in range(_DREGS):
            zbuf[r, pl.ds(j * _LANES, _LANES)] = zero16
        return 0

    lax.fori_loop(0, _CH, zrow, 0, unroll=False)

    def part(g, carry):
        n1, n0 = carry
        id16 = mslot2[g, :]
        m16 = uslot2[g, :]
        slot0 = base + g * _LANES
        for t in range(_LANES):
            mt = m16[t]
            ids_c[pl.ds(n1, _LANES)] = jnp.full((_LANES,), id16[t], jnp.int32)
            uslot_c[pl.ds(n1, _LANES)] = jnp.full((_LANES,), slot0 + t,
                                                  jnp.int32)
            mslot_c[pl.ds(n0, _LANES)] = jnp.full((_LANES,), slot0 + t,
                                                  jnp.int32)
            n1 = n1 + mt
            n0 = n0 + (1 - mt)
        return n1, n0

    def gdesc(k):
        return pltpu.make_async_copy(
            table_hbm.at[ids_c.at[pl.ds(k * _CH, _CH)]], rbuf.at[k % _NB],
            gsems.at[k % _NB])

    def sdesc(k, b):
        return pltpu.make_async_copy(rbuf.at[b], out_hbm.at[uslot2.at[k]],
                                     osems.at[b])

    def relay_u(lo, hi):
        def r_(r, _):
            uslot2[r, :] = uslot_c[pl.ds(r * _LANES, _LANES)]
            return 0
        lax.fori_loop(lo, hi, r_, 0, unroll=False)

    def relay_m(lo, hi):
        def r_(r, _):
            mslot2[r, :] = mslot_c[pl.ds(r * _LANES, _LANES)]
            return 0
        lax.fori_loop(lo, hi, r_, 0, unroll=False)

    def zdesc(k):
        return pltpu.make_async_copy(zbuf, out_hbm.at[mslot2.at[k]], zsem)

    def issue_z(lo, hi):
        def z_(k, _):
            zdesc(k).start()

            @pl.when(k >= _ZWIN)
            def _():
                zdesc(0).wait()
            return 0
        lax.fori_loop(lo, hi, z_, 0, unroll=False)

    def start_gather(k):
        @pl.when(k >= _NB)
        def _():
            sdesc(0, k % _NB).wait()

        gdesc(k).start()

    def issue_u(lo, hi):
        def pref(i, _):
            k = lo + i

            @pl.when(k < hi)
            def _():
                start_gather(k)
            return 0

        lax.fori_loop(0, _GDIST, pref, 0, unroll=False)

        def u_(k, _):
            gdesc(k).wait()
            sdesc(k, k % _NB).start()
            kp = k + _GDIST

            @pl.when(kp < hi)
            def _():
                start_gather(kp)
            return 0

        lax.fori_loop(lo, hi, u_, 0, unroll=False)

    def block(j, carry):
        n1, n0, uf, mf = carry
        n1, n0 = lax.fori_loop(j * _BLK, (j + 1) * _BLK, part, (n1, n0),
                               unroll=False)
        ufn = n1 // _CH
        mfn = n0 // _CH
        relay_u(uf, ufn)
        relay_m(mf, mfn)
        issue_z(mf, mfn)
        issue_u(uf, ufn)
        return n1, n0, ufn, mfn

    z = jnp.int32(0)
    n1, n0, uf, mf = lax.fori_loop(0, _NBLK, block, (z, z, z, z),
                                   unroll=False)

    @pl.when(n1 > 0)
    def _():
        last_id = ids_c[pl.ds(n1 - 1, _LANES)][0]
        last_sl = uslot_c[pl.ds(n1 - 1, _LANES)][0]
        ids_c[pl.ds(n1, _LANES)] = jnp.full((_LANES,), last_id, jnp.int32)
        uslot_c[pl.ds(n1, _LANES)] = jnp.full((_LANES,), last_sl, jnp.int32)

    @pl.when(n0 > 0)
    def _():
        last_ms = mslot_c[pl.ds(n0 - 1, _LANES)][0]
        mslot_c[pl.ds(n0, _LANES)] = jnp.full((_LANES,), last_ms, jnp.int32)

    c1 = (n1 + _CH - 1) // _CH
    c0 = (n0 + _CH - 1) // _CH
    relay_u(uf, c1)
    relay_m(mf, c0)
    issue_z(mf, c0)
    issue_u(uf, c1)

    def dr(i, _):
        k = jnp.maximum(c1 - _NB, 0) + i

        @pl.when(k < c1)
        def _():
            sdesc(0, k % _NB).wait()
        return 0

    lax.fori_loop(0, _NB, dr, 0, unroll=False)

    lax.fori_loop(0, jnp.minimum(c0, _ZWIN),
                  lambda i, _: (zdesc(0).wait(), 0)[1], 0, unroll=False)


@jax.jit
def _lookup(ids, mask_i, table):
    mesh = plsc.VectorSubcoreMesh(core_axis_name="c", subcore_axis_name="s")
    run = pl.kernel(
        _body,
        out_type=jax.ShapeDtypeStruct((_TOK, _DIM), jnp.float32),
        mesh=mesh,
        scratch_types=[
            pltpu.VMEM((_BUF,), jnp.int32),
            pltpu.VMEM((_BUF,), jnp.int32),
            pltpu.VMEM((_BUF,), jnp.int32),
            pltpu.VMEM((_ROWS, _CH), jnp.int32),
            pltpu.VMEM((_ROWS, _CH), jnp.int32),
            pltpu.VMEM((_NB, _CH, _DIM), jnp.float32),
            pltpu.VMEM((_CH, _DIM), jnp.float32),
            pltpu.SemaphoreType.DMA((_NB,)),
            pltpu.SemaphoreType.DMA((_NB,)),
            pltpu.SemaphoreType.DMA,
        ],
    )
    return run(ids, mask_i, table)


def kernel(input_ids, attention_mask, table):
    ids = input_ids.reshape(_NW, _GRP, _LANES).astype(jnp.int32)
    mask_i = attention_mask.reshape(_NW, _GRP, _LANES).astype(jnp.int32)
    out = _lookup(ids, mask_i, table)
    return out.reshape(_BATCH, _SEQ, _DIM), attention_mask


# speedup vs baseline: 1.0432x; 1.0082x over previous
import jax
import jax.numpy as jnp
from jax import lax
from jax.experimental import pallas as pl
from jax.experimental.pallas import tpu as pltpu
from jax.experimental.pallas import tpu_sc as plsc

_VOCAB = 50257
_DIM = 768
_BATCH = 64
_SEQ = 1024
_TOK = _BATCH * _SEQ

_NC = 2
_NS = 16
_NW = _NC * _NS
_TPW = _TOK // _NW
_LANES = 16
_CH = _LANES
_GRP = _TPW // _LANES
_ROWS = _GRP + 2
_BUF = _ROWS * _LANES
_DREGS = _DIM // _LANES
_ZWIN = 16
_NB = 4
_GDIST = 2
_BLK = 8
_NBLK = _GRP // _BLK


def _body(ids_hbm, mask_hbm, table_hbm, out_hbm,
          ids_c, uslot_c, mslot_c, uslot2, mslot2,
          rbuf, zbuf, gsems, osems, zsem):
    wid = lax.axis_index("c") * _NS + lax.axis_index("s")
    base = wid * _TPW
    zero16 = jnp.zeros((_LANES,), jnp.float32)

    pltpu.sync_copy(ids_hbm.at[wid], mslot2.at[pl.ds(0, _GRP)])
    pltpu.sync_copy(mask_hbm.at[wid], uslot2.at[pl.ds(0, _GRP)])

    def zrow(r, _):
        for j in range(_DREGS):
            zbuf[r, pl.ds(j * _LANES, _LANES)] = zero16
        return 0

    lax.fori_loop(0, _CH, zrow, 0, unroll=False)

    def part(g, carry):
        n1, n0 = carry
        id16 = mslot2[g, :]
        m16 = uslot2[g, :]
        slot0 = base + g * _LANES
        for t in range(_LANES):
            mt = m16[t]
            ids_c[pl.ds(n1, _LANES)] = jnp.full((_LANES,), id16[t], jnp.int32)
            uslot_c[pl.ds(n1, _LANES)] = jnp.full((_LANES,), slot0 + t,
                                                  jnp.int32)
            mslot_c[pl.ds(n0, _LANES)] = jnp.full((_LANES,), slot0 + t,
                                                  jnp.int32)
            n1 = n1 + mt
            n0 = n0 + (1 - mt)
        return n1, n0

    def gdesc(k):
        return pltpu.make_async_copy(
            table_hbm.at[ids_c.at[pl.ds(k * _CH, _CH)]], rbuf.at[k % _NB],
            gsems.at[k % _NB])

    def sdesc(k, b):
        return pltpu.make_async_copy(rbuf.at[b], out_hbm.at[uslot2.at[k]],
                                     osems.at[b])

    def relay_u(lo, hi):
        def r_(r, _):
            uslot2[r, :] = uslot_c[pl.ds(r * _LANES, _LANES)]
            return 0
        lax.fori_loop(lo, hi, r_, 0, unroll=False)

    def relay_m(lo, hi):
        def r_(r, _):
            mslot2[r, :] = mslot_c[pl.ds(r * _LANES, _LANES)]
            return 0
        lax.fori_loop(lo, hi, r_, 0, unroll=False)

    def zdesc(k):
        return pltpu.make_async_copy(zbuf, out_hbm.at[mslot2.at[k]], zsem)

    def issue_z(lo, hi):
        def z_(k, _):
            zdesc(k).start()

            @pl.when(k >= _ZWIN)
            def _():
                zdesc(0).wait()
            return 0
        lax.fori_loop(lo, hi, z_, 0, unroll=False)

    def start_gather(k):
        @pl.when(k >= _NB)
        def _():
            sdesc(0, k % _NB).wait()

        gdesc(k).start()

    def issue_u(lo, hi):
        def pref(i, _):
            k = lo + i

            @pl.when(k < hi)
            def _():
                start_gather(k)
            return 0

        lax.fori_loop(0, _GDIST, pref, 0, unroll=False)

        def u_(k, _):
            gdesc(k).wait()
            sdesc(k, k % _NB).start()
            kp = k + _GDIST

            @pl.when(kp < hi)
            def _():
                start_gather(kp)
            return 0

        lax.fori_loop(lo, hi, u_, 0, unroll=False)

    def block(j, carry):
        n1, n0, uf, mf = carry
        n1, n0 = lax.fori_loop(j * _BLK, (j + 1) * _BLK, part, (n1, n0),
                               unroll=False)
        ufn = n1 // _CH
        mfn = n0 // _CH
        relay_u(uf, ufn)
        relay_m(mf, mfn)
        issue_z(mf, mfn)
        issue_u(uf, ufn)
        return n1, n0, ufn, mfn

    z = jnp.int32(0)
    n1, n0, uf, mf = lax.fori_loop(0, _NBLK, block, (z, z, z, z),
                                   unroll=False)

    @pl.when(n1 > 0)
    def _():
        last_id = ids_c[pl.ds(n1 - 1, _LANES)][0]
        last_sl = uslot_c[pl.ds(n1 - 1, _LANES)][0]
        ids_c[pl.ds(n1, _LANES)] = jnp.full((_LANES,), last_id, jnp.int32)
        uslot_c[pl.ds(n1, _LANES)] = jnp.full((_LANES,), last_sl, jnp.int32)

    @pl.when(n0 > 0)
    def _():
        last_ms = mslot_c[pl.ds(n0 - 1, _LANES)][0]
        mslot_c[pl.ds(n0, _LANES)] = jnp.full((_LANES,), last_ms, jnp.int32)

    c1 = (n1 + _CH - 1) // _CH
    c0 = (n0 + _CH - 1) // _CH
    relay_u(uf, c1)
    relay_m(mf, c0)
    issue_z(mf, c0)
    issue_u(uf, c1)

    def dr(i, _):
        k = jnp.maximum(c1 - _NB, 0) + i

        @pl.when(k < c1)
        def _():
            sdesc(0, k % _NB).wait()
        return 0

    lax.fori_loop(0, _NB, dr, 0, unroll=False)

    lax.fori_loop(0, jnp.minimum(c0, _ZWIN),
                  lambda i, _: (zdesc(0).wait(), 0)[1], 0, unroll=False)


@jax.jit
def _lookup(ids, mask_i, table):
    mesh = plsc.VectorSubcoreMesh(core_axis_name="c", subcore_axis_name="s")
    run = pl.kernel(
        _body,
        out_type=jax.ShapeDtypeStruct((_TOK, _DIM), jnp.float32),
        mesh=mesh,
        scratch_types=[
            pltpu.VMEM((_BUF,), jnp.int32),
            pltpu.VMEM((_BUF,), jnp.int32),
            pltpu.VMEM((_BUF,), jnp.int32),
            pltpu.VMEM((_ROWS, _CH), jnp.int32),
            pltpu.VMEM((_ROWS, _CH), jnp.int32),
            pltpu.VMEM((_NB, _CH, _DIM), jnp.float32),
            pltpu.VMEM((_CH, _DIM), jnp.float32),
            pltpu.SemaphoreType.DMA((_NB,)),
            pltpu.SemaphoreType.DMA((_NB,)),
            pltpu.SemaphoreType.DMA,
        ],
    )
    return run(ids, mask_i, table)


def kernel(input_ids, attention_mask, table):
    ids = input_ids.reshape(_NW, _GRP, _LANES).astype(jnp.int32)
    mask_i = attention_mask.reshape(_NW, _GRP, _LANES).astype(jnp.int32)
    out = _lookup(ids, mask_i, table)
    return out.reshape(_BATCH, _SEQ, _DIM), attention_mask
